# Initial kernel scaffold; baseline (speedup 1.0000x reference)
import jax
import jax.numpy as jnp
from jax.experimental import pallas as pl


def kernel(x, edge_index, batch, W_l1, b_l1, W_r1, W_l2, b_l2, W_r2, W_l3, b_l3, W_r3, W_lin, b_lin):
    raise NotImplementedError("write your pallas kernel here")


# speedup vs baseline: 5.9506x; 5.9506x over previous
import functools

import jax
import jax.numpy as jnp
from jax import lax
from jax.experimental import pallas as pl
from jax.experimental.pallas import tpu as pltpu
from jax.experimental.pallas import tpu_sc as plsc

_NC = 2
_NS = 16
_G = 32
_CW = 16


def _largest_divisor(n, cap):
    for c in range(cap, 0, -1):
        if n % c == 0 and c % 8 == 0:
            return c
    return None


def _sc_aggregate(y, src, dst, with_cnt):
    n, w = y.shape
    e = src.shape[0]
    nw = _NC * _NS
    assert e % nw == 0
    ept = e // nw
    ch = _largest_divisor(ept, 128)
    nchunk = ept // ch
    npad = -(-n // (8 * _NS)) * (8 * _NS)
    rows_pt = npad // _NS
    zrows = 1
    for c in range(min(rows_pt, 128), 0, -1):
        if rows_pt % c == 0:
            zrows = c
            break
    nz = rows_pt // zrows

    mesh = plsc.VectorSubcoreMesh(core_axis_name="c", subcore_axis_name="s")
    out_type = [jax.ShapeDtypeStruct((_NC, npad, w), jnp.float32)]
    scratch = [
        pltpu.VMEM((ch,), jnp.int32),
        pltpu.VMEM((ch,), jnp.int32),
        pltpu.VMEM((ch, w), jnp.float32),
        pltpu.VMEM((zrows, w), jnp.float32),
        pltpu.VMEM_SHARED((npad, w), jnp.float32),
        pltpu.SemaphoreType.DMA,
    ]
    if with_cnt:
        out_type.append(jax.ShapeDtypeStruct((_NC * npad,), jnp.float32))
        scratch += [
            pltpu.VMEM((ch,), jnp.float32),
            pltpu.VMEM((rows_pt,), jnp.float32),
            pltpu.VMEM_SHARED((npad,), jnp.float32),
        ]

    def body(y_hbm, src_hbm, dst_hbm, *rest):
        if with_cnt:
            (acc_out, cnt_out, src_v, dst_v, rows_v, z_v, acc_s, sem,
             ones_v, zc_v, cnt_s) = rest
        else:
            acc_out, src_v, dst_v, rows_v, z_v, acc_s, sem = rest
        cid = lax.axis_index("c")
        sid = lax.axis_index("s")
        wid = sid * _NC + cid

        wv = w // 16

        def zfill(k, _):
            i = k // wv
            j = k % wv
            z_v[i, pl.ds(j * 16, 16)] = jnp.zeros((16,), jnp.float32)
            return 0

        lax.fori_loop(0, zrows * wv, zfill, 0)
        r0 = sid * rows_pt
        for k in range(nz):
            pltpu.sync_copy(z_v, acc_s.at[pl.ds(r0 + k * zrows, zrows)])
        if with_cnt:
            def onesfill(j, _):
                ones_v[pl.ds(j * 16, 16)] = jnp.ones((16,), jnp.float32)
                return 0

            lax.fori_loop(0, ch // 16, onesfill, 0)

            def zcfill(j, _):
                zc_v[pl.ds(j * 16, 16)] = jnp.zeros((16,), jnp.float32)
                return 0

            lax.fori_loop(0, rows_pt // 16, zcfill, 0)
            pltpu.sync_copy(zc_v, cnt_s.at[pl.ds(r0, rows_pt)])
        plsc.subcore_barrier()

        ebase = wid * ept

        def step(c, _):
            off = ebase + c * ch
            pltpu.sync_copy(src_hbm.at[pl.ds(off, ch)], src_v)
            pltpu.sync_copy(dst_hbm.at[pl.ds(off, ch)], dst_v)
            pltpu.async_copy(y_hbm.at[src_v], rows_v, sem).wait()
            pltpu.sync_copy(rows_v, acc_s.at[dst_v], add=True)
            if with_cnt:
                pltpu.sync_copy(ones_v, cnt_s.at[dst_v], add=True)
            return 0

        lax.fori_loop(0, nchunk, step, 0)
        plsc.subcore_barrier()

        pltpu.sync_copy(acc_s.at[pl.ds(r0, rows_pt)],
                        acc_out.at[cid, pl.ds(r0, rows_pt)])
        if with_cnt:
            pltpu.sync_copy(cnt_s.at[pl.ds(r0, rows_pt)], zc_v)
            pltpu.sync_copy(zc_v, cnt_out.at[pl.ds(cid * npad + r0, rows_pt)])

    fn = pl.kernel(body, out_type=out_type, mesh=mesh, scratch_types=scratch)
    return fn(y, src, dst)


def _dot(a, b):
    return jnp.dot(a, b, preferred_element_type=jnp.float32)


def _mean_from_partials(p_ref, c_ref):
    psum = p_ref[0] + p_ref[1]
    cnt = jnp.maximum(c_ref[0] + c_ref[1], 1.0)
    return psum / cnt


def _tc_layer1(agg, cntp, x, W_l1, b_l1, W_r1, W_l2, b_l2, W_r2, interpret=False):
    n, d = x.shape
    k1 = W_l1.shape[1]
    k2 = W_l2.shape[1]
    R = 1000
    grid = (n // R,)

    def body(p_ref, c_ref, x_ref, wl1, bl1, wr1, wl2, bl2, wr2, y2_ref, s2_ref):
        mean = _mean_from_partials(p_ref, c_ref)
        h1 = jnp.maximum(
            _dot(mean, wl1[...]) + bl1[...] + _dot(x_ref[...], wr1[...]), 0.0)
        y2_ref[...] = _dot(h1, wl2[...])
        s2_ref[...] = _dot(h1, wr2[...]) + bl2[...]

    return pl.pallas_call(
        body,
        grid=grid,
        in_specs=[
            pl.BlockSpec((_NC, R, d), lambda i: (0, i, 0)),
            pl.BlockSpec((_NC, R, 1), lambda i: (0, i, 0)),
            pl.BlockSpec((R, d), lambda i: (i, 0)),
            pl.BlockSpec((d, k1), lambda i: (0, 0)),
            pl.BlockSpec((1, k1), lambda i: (0, 0)),
            pl.BlockSpec((d, k1), lambda i: (0, 0)),
            pl.BlockSpec((k1, k2), lambda i: (0, 0)),
            pl.BlockSpec((1, k2), lambda i: (0, 0)),
            pl.BlockSpec((k1, k2), lambda i: (0, 0)),
        ],
        out_specs=[
            pl.BlockSpec((R, k2), lambda i: (i, 0)),
            pl.BlockSpec((R, k2), lambda i: (i, 0)),
        ],
        out_shape=[
            jax.ShapeDtypeStruct((n, k2), jnp.float32),
            jax.ShapeDtypeStruct((n, k2), jnp.float32),
        ],
        interpret=interpret,
    )(agg, cntp, x, W_l1, b_l1.reshape(1, -1), W_r1, W_l2,
      b_l2.reshape(1, -1), W_r2)


def _tc_layer2(agg, cntp, s2, Wcat, bcat, interpret=False):
    n, d = s2.shape
    k2 = Wcat.shape[1]
    R = 1000
    grid = (n // R,)

    def body(p_ref, c_ref, s2_ref, wcat, bc, ycat_ref):
        mean = _mean_from_partials(p_ref, c_ref)
        h2 = jnp.maximum(mean + s2_ref[...], 0.0)
        ycat_ref[...] = _dot(h2, wcat[...]) + bc[...]

    return pl.pallas_call(
        body,
        grid=grid,
        in_specs=[
            pl.BlockSpec((_NC, R, d), lambda i: (0, i, 0)),
            pl.BlockSpec((_NC, R, 1), lambda i: (0, i, 0)),
            pl.BlockSpec((R, d), lambda i: (i, 0)),
            pl.BlockSpec((d, k2), lambda i: (0, 0)),
            pl.BlockSpec((1, k2), lambda i: (0, 0)),
        ],
        out_specs=pl.BlockSpec((R, k2), lambda i: (i, 0)),
        out_shape=jax.ShapeDtypeStruct((n, k2), jnp.float32),
        interpret=interpret,
    )(agg, cntp, s2, Wcat, bcat)


def _tc_layer3(agg, cntp, ycat, batch2d, W_lin, b_lin, interpret=False):
    n, dc = ycat.shape
    d = dc // 2
    out = W_lin.shape[1]
    R = 1000
    grid = (n // R,)
    last = grid[0] - 1

    def body(p_ref, c_ref, y_ref, b_ref, wlin, blin, out_ref, acc, accg):
        i = pl.program_id(0)

        @pl.when(i == 0)
        def _():
            acc[...] = jnp.zeros_like(acc)
            accg[...] = jnp.zeros_like(accg)

        psum = p_ref[0] + p_ref[1]
        cnt = jnp.maximum(c_ref[0] + c_ref[1], 1.0)
        mean = psum[:, :d] / cnt
        h3 = jnp.maximum(mean + y_ref[...][:, d:], 0.0)
        onehot = (b_ref[...] == lax.broadcasted_iota(jnp.int32, (R, _G), 1)
                  ).astype(jnp.float32)
        acc[...] += lax.dot_general(onehot, h3, (((0,), (0,)), ((), ())),
                                    preferred_element_type=jnp.float32)
        accg[...] += lax.dot_general(
            onehot, jnp.ones((R, 128), jnp.float32), (((0,), (0,)), ((), ())),
            preferred_element_type=jnp.float32)

        @pl.when(i == last)
        def _():
            pooled = acc[...] / jnp.maximum(accg[...][:, 0:1], 1.0)
            logits = _dot(pooled, wlin[...]) + blin[...]
            m = jnp.max(logits, axis=1, keepdims=True)
            lse = jnp.log(jnp.sum(jnp.exp(logits - m), axis=1, keepdims=True))
            out_ref[...] = logits - m - lse

    return pl.pallas_call(
        body,
        grid=grid,
        in_specs=[
            pl.BlockSpec((_NC, R, dc), lambda i: (0, i, 0)),
            pl.BlockSpec((_NC, R, 1), lambda i: (0, i, 0)),
            pl.BlockSpec((R, dc), lambda i: (i, 0)),
            pl.BlockSpec((R, 1), lambda i: (i, 0)),
            pl.BlockSpec((d, out), lambda i: (0, 0)),
            pl.BlockSpec((1, out), lambda i: (0, 0)),
        ],
        out_specs=pl.BlockSpec((_G, out), lambda i: (0, 0)),
        out_shape=jax.ShapeDtypeStruct((_G, out), jnp.float32),
        scratch_shapes=[
            pltpu.VMEM((_G, d), jnp.float32),
            pltpu.VMEM((_G, 128), jnp.float32),
        ],
        interpret=interpret,
    )(agg, cntp, ycat, batch2d, W_lin, b_lin.reshape(1, -1))


def kernel(x, edge_index, batch, W_l1, b_l1, W_r1, W_l2, b_l2, W_r2,
           W_l3, b_l3, W_r3, W_lin, b_lin):
    src = edge_index[0]
    dst = edge_index[1]
    agg1, cntp = _sc_aggregate(x, src, dst, with_cnt=True)
    cntp = cntp.reshape(_NC, -1, 1)
    y2, s2 = _tc_layer1(agg1, cntp, x, W_l1, b_l1, W_r1, W_l2, b_l2, W_r2)
    agg2 = _sc_aggregate(y2, src, dst, with_cnt=False)[0]
    Wcat = jnp.concatenate([W_l3, W_r3], axis=1)
    bcat = jnp.concatenate(
        [jnp.zeros_like(b_l3), b_l3]).reshape(1, -1)
    ycat = _tc_layer2(agg2, cntp, s2, Wcat, bcat)
    agg3 = _sc_aggregate(ycat, src, dst, with_cnt=False)[0]
    return _tc_layer3(agg3, cntp, ycat, batch.reshape(-1, 1), W_lin, b_lin)


# speedup vs baseline: 11.5569x; 1.9421x over previous
import functools

import jax
import jax.numpy as jnp
from jax import lax
from jax.experimental import pallas as pl
from jax.experimental.pallas import tpu as pltpu
from jax.experimental.pallas import tpu_sc as plsc

_NC = 2
_NS = 16
_G = 32
_CW = 16


def _largest_divisor(n, cap):
    for c in range(cap, 0, -1):
        if n % c == 0 and c % 8 == 0:
            return c
    return None


def _sc_aggregate(y, src, dst, with_cnt):
    n, w = y.shape
    e = src.shape[0]
    nw = _NC * _NS
    assert e % nw == 0
    ept = e // nw
    ch = _largest_divisor(ept, 128)
    nchunk = ept // ch
    npad = -(-n // (8 * _NS)) * (8 * _NS)
    rows_pt = npad // _NS
    zrows = 1
    for c in range(min(rows_pt, 128), 0, -1):
        if rows_pt % c == 0:
            zrows = c
            break
    nz = rows_pt // zrows

    mesh = plsc.VectorSubcoreMesh(core_axis_name="c", subcore_axis_name="s")
    out_type = [jax.ShapeDtypeStruct((_NC, npad, w), jnp.float32)]
    assert nchunk % 2 == 1 and nchunk >= 3
    scratch = [
        pltpu.VMEM((ch,), jnp.int32),
        pltpu.VMEM((ch,), jnp.int32),
        pltpu.VMEM((ch,), jnp.int32),
        pltpu.VMEM((ch,), jnp.int32),
        pltpu.VMEM((ch, w), jnp.float32),
        pltpu.VMEM((ch, w), jnp.float32),
        pltpu.VMEM((zrows, w), jnp.float32),
        pltpu.VMEM_SHARED((npad, w), jnp.float32),
        pltpu.SemaphoreType.DMA,
        pltpu.SemaphoreType.DMA,
        pltpu.SemaphoreType.DMA,
        pltpu.SemaphoreType.DMA,
    ]
    if with_cnt:
        out_type.append(jax.ShapeDtypeStruct((_NC * npad,), jnp.float32))
        scratch += [
            pltpu.VMEM((ch,), jnp.float32),
            pltpu.VMEM((rows_pt,), jnp.float32),
            pltpu.VMEM_SHARED((npad,), jnp.float32),
        ]

    def body(y_hbm, src_hbm, dst_hbm, *rest):
        if with_cnt:
            (acc_out, cnt_out, src0, dst0, src1, dst1, rows0, rows1, z_v,
             acc_s, semg0, semg1, semi0, semi1, ones_v, zc_v, cnt_s) = rest
        else:
            (acc_out, src0, dst0, src1, dst1, rows0, rows1, z_v, acc_s,
             semg0, semg1, semi0, semi1) = rest
        cid = lax.axis_index("c")
        sid = lax.axis_index("s")
        wid = sid * _NC + cid

        wv = w // 16

        def zfill(k, _):
            i = k // wv
            j = k % wv
            z_v[i, pl.ds(j * 16, 16)] = jnp.zeros((16,), jnp.float32)
            return 0

        lax.fori_loop(0, zrows * wv, zfill, 0)
        r0 = sid * rows_pt
        for k in range(nz):
            pltpu.sync_copy(z_v, acc_s.at[pl.ds(r0 + k * zrows, zrows)])
        if with_cnt:
            def onesfill(j, _):
                ones_v[pl.ds(j * 16, 16)] = jnp.ones((16,), jnp.float32)
                return 0

            lax.fori_loop(0, ch // 16, onesfill, 0)

            def zcfill(j, _):
                zc_v[pl.ds(j * 16, 16)] = jnp.zeros((16,), jnp.float32)
                return 0

            lax.fori_loop(0, rows_pt // 16, zcfill, 0)
            pltpu.sync_copy(zc_v, cnt_s.at[pl.ds(r0, rows_pt)])
        plsc.subcore_barrier()

        ebase = wid * ept

        def iload(j, sv, dv, sem):
            off = ebase + j * ch
            pltpu.async_copy(src_hbm.at[pl.ds(off, ch)], sv, sem)
            pltpu.async_copy(dst_hbm.at[pl.ds(off, ch)], dv, sem)

        def iwait(j, sv, dv, sem):
            off = ebase + j * ch
            pltpu.make_async_copy(src_hbm.at[pl.ds(off, ch)], sv, sem).wait()
            pltpu.make_async_copy(dst_hbm.at[pl.ds(off, ch)], dv, sem).wait()

        def gath(sv, buf, sem):
            pltpu.async_copy(y_hbm.at[sv], buf, sem)

        def gwait(sv, buf, sem):
            pltpu.make_async_copy(y_hbm.at[sv], buf, sem).wait()

        def scat(dv, buf):
            pltpu.sync_copy(buf, acc_s.at[dv], add=True)
            if with_cnt:
                pltpu.sync_copy(ones_v, cnt_s.at[dv], add=True)

        iload(0, src0, dst0, semi0)
        iwait(0, src0, dst0, semi0)
        gath(src0, rows0, semg0)
        iload(1, src1, dst1, semi1)

        def pair(k, _):
            j = 2 * k
            iwait(j + 1, src1, dst1, semi1)
            gath(src1, rows1, semg1)
            gwait(src0, rows0, semg0)
            scat(dst0, rows0)
            iload(j + 2, src0, dst0, semi0)
            iwait(j + 2, src0, dst0, semi0)
            gath(src0, rows0, semg0)
            gwait(src1, rows1, semg1)
            scat(dst1, rows1)

            @pl.when(j + 3 < nchunk)
            def _():
                iload(j + 3, src1, dst1, semi1)
            return 0

        lax.fori_loop(0, (nchunk - 1) // 2, pair, 0)
        gwait(src0, rows0, semg0)
        scat(dst0, rows0)
        plsc.subcore_barrier()

        pltpu.sync_copy(acc_s.at[pl.ds(r0, rows_pt)],
                        acc_out.at[cid, pl.ds(r0, rows_pt)])
        if with_cnt:
            pltpu.sync_copy(cnt_s.at[pl.ds(r0, rows_pt)], zc_v)
            pltpu.sync_copy(zc_v, cnt_out.at[pl.ds(cid * npad + r0, rows_pt)])

    fn = pl.kernel(body, out_type=out_type, mesh=mesh, scratch_types=scratch)
    return fn(y, src, dst)


def _dot(a, b):
    return jnp.dot(a, b, preferred_element_type=jnp.float32)


def _mean_from_partials(p_ref, c_ref):
    psum = p_ref[0] + p_ref[1]
    cnt = jnp.maximum(c_ref[0] + c_ref[1], 1.0)
    return psum / cnt


def _tc_layer1(agg, cntp, x, W_l1, b_l1, W_r1, W_l2, b_l2, W_r2, interpret=False):
    n, d = x.shape
    k1 = W_l1.shape[1]
    k2 = W_l2.shape[1]
    R = 1000
    grid = (n // R,)

    def body(p_ref, c_ref, x_ref, wl1, bl1, wr1, wl2, bl2, wr2, y2_ref, s2_ref):
        mean = _mean_from_partials(p_ref, c_ref)
        h1 = jnp.maximum(
            _dot(mean, wl1[...]) + bl1[...] + _dot(x_ref[...], wr1[...]), 0.0)
        y2_ref[...] = _dot(h1, wl2[...])
        s2_ref[...] = _dot(h1, wr2[...]) + bl2[...]

    return pl.pallas_call(
        body,
        grid=grid,
        in_specs=[
            pl.BlockSpec((_NC, R, d), lambda i: (0, i, 0)),
            pl.BlockSpec((_NC, R, 1), lambda i: (0, i, 0)),
            pl.BlockSpec((R, d), lambda i: (i, 0)),
            pl.BlockSpec((d, k1), lambda i: (0, 0)),
            pl.BlockSpec((1, k1), lambda i: (0, 0)),
            pl.BlockSpec((d, k1), lambda i: (0, 0)),
            pl.BlockSpec((k1, k2), lambda i: (0, 0)),
            pl.BlockSpec((1, k2), lambda i: (0, 0)),
            pl.BlockSpec((k1, k2), lambda i: (0, 0)),
        ],
        out_specs=[
            pl.BlockSpec((R, k2), lambda i: (i, 0)),
            pl.BlockSpec((R, k2), lambda i: (i, 0)),
        ],
        out_shape=[
            jax.ShapeDtypeStruct((n, k2), jnp.float32),
            jax.ShapeDtypeStruct((n, k2), jnp.float32),
        ],
        interpret=interpret,
    )(agg, cntp, x, W_l1, b_l1.reshape(1, -1), W_r1, W_l2,
      b_l2.reshape(1, -1), W_r2)


def _tc_layer2(agg, cntp, s2, Wcat, bcat, interpret=False):
    n, d = s2.shape
    k2 = Wcat.shape[1]
    R = 1000
    grid = (n // R,)

    def body(p_ref, c_ref, s2_ref, wcat, bc, ycat_ref):
        mean = _mean_from_partials(p_ref, c_ref)
        h2 = jnp.maximum(mean + s2_ref[...], 0.0)
        ycat_ref[...] = _dot(h2, wcat[...]) + bc[...]

    return pl.pallas_call(
        body,
        grid=grid,
        in_specs=[
            pl.BlockSpec((_NC, R, d), lambda i: (0, i, 0)),
            pl.BlockSpec((_NC, R, 1), lambda i: (0, i, 0)),
            pl.BlockSpec((R, d), lambda i: (i, 0)),
            pl.BlockSpec((d, k2), lambda i: (0, 0)),
            pl.BlockSpec((1, k2), lambda i: (0, 0)),
        ],
        out_specs=pl.BlockSpec((R, k2), lambda i: (i, 0)),
        out_shape=jax.ShapeDtypeStruct((n, k2), jnp.float32),
        interpret=interpret,
    )(agg, cntp, s2, Wcat, bcat)


def _tc_layer3(agg, cntp, ycat, batch2d, W_lin, b_lin, interpret=False):
    n, dc = ycat.shape
    d = dc // 2
    out = W_lin.shape[1]
    R = 1000
    grid = (n // R,)
    last = grid[0] - 1

    def body(p_ref, c_ref, y_ref, b_ref, wlin, blin, out_ref, acc, accg):
        i = pl.program_id(0)

        @pl.when(i == 0)
        def _():
            acc[...] = jnp.zeros_like(acc)
            accg[...] = jnp.zeros_like(accg)

        psum = p_ref[0] + p_ref[1]
        cnt = jnp.maximum(c_ref[0] + c_ref[1], 1.0)
        mean = psum[:, :d] / cnt
        h3 = jnp.maximum(mean + y_ref[...][:, d:], 0.0)
        onehot = (b_ref[...] == lax.broadcasted_iota(jnp.int32, (R, _G), 1)
                  ).astype(jnp.float32)
        acc[...] += lax.dot_general(onehot, h3, (((0,), (0,)), ((), ())),
                                    preferred_element_type=jnp.float32)
        accg[...] += lax.dot_general(
            onehot, jnp.ones((R, 128), jnp.float32), (((0,), (0,)), ((), ())),
            preferred_element_type=jnp.float32)

        @pl.when(i == last)
        def _():
            pooled = acc[...] / jnp.maximum(accg[...][:, 0:1], 1.0)
            logits = _dot(pooled, wlin[...]) + blin[...]
            m = jnp.max(logits, axis=1, keepdims=True)
            lse = jnp.log(jnp.sum(jnp.exp(logits - m), axis=1, keepdims=True))
            out_ref[...] = logits - m - lse

    return pl.pallas_call(
        body,
        grid=grid,
        in_specs=[
            pl.BlockSpec((_NC, R, dc), lambda i: (0, i, 0)),
            pl.BlockSpec((_NC, R, 1), lambda i: (0, i, 0)),
            pl.BlockSpec((R, dc), lambda i: (i, 0)),
            pl.BlockSpec((R, 1), lambda i: (i, 0)),
            pl.BlockSpec((d, out), lambda i: (0, 0)),
            pl.BlockSpec((1, out), lambda i: (0, 0)),
        ],
        out_specs=pl.BlockSpec((_G, out), lambda i: (0, 0)),
        out_shape=jax.ShapeDtypeStruct((_G, out), jnp.float32),
        scratch_shapes=[
            pltpu.VMEM((_G, d), jnp.float32),
            pltpu.VMEM((_G, 128), jnp.float32),
        ],
        interpret=interpret,
    )(agg, cntp, ycat, batch2d, W_lin, b_lin.reshape(1, -1))


def kernel(x, edge_index, batch, W_l1, b_l1, W_r1, W_l2, b_l2, W_r2,
           W_l3, b_l3, W_r3, W_lin, b_lin):
    src = edge_index[0]
    dst = edge_index[1]
    agg1, cntp = _sc_aggregate(x, src, dst, with_cnt=True)
    cntp = cntp.reshape(_NC, -1, 1)
    y2, s2 = _tc_layer1(agg1, cntp, x, W_l1, b_l1, W_r1, W_l2, b_l2, W_r2)
    agg2 = _sc_aggregate(y2, src, dst, with_cnt=False)[0]
    Wcat = jnp.concatenate([W_l3, W_r3], axis=1)
    bcat = jnp.concatenate(
        [jnp.zeros_like(b_l3), b_l3]).reshape(1, -1)
    ycat = _tc_layer2(agg2, cntp, s2, Wcat, bcat)
    agg3 = _sc_aggregate(ycat, src, dst, with_cnt=False)[0]
    return _tc_layer3(agg3, cntp, ycat, batch.reshape(-1, 1), W_lin, b_lin)


# speedup vs baseline: 13.6685x; 1.1827x over previous
import functools

import jax
import jax.numpy as jnp
from jax import lax
from jax.experimental import pallas as pl
from jax.experimental.pallas import tpu as pltpu
from jax.experimental.pallas import tpu_sc as plsc

_NC = 2
_NS = 16
_G = 32
_CW = 16


def _largest_divisor(n, cap):
    for c in range(cap, 0, -1):
        if n % c == 0 and c % 8 == 0:
            return c
    return None


def _sc_aggregate(y, src, dst, with_cnt):
    n, w = y.shape
    e = src.shape[0]
    nw = _NC * _NS
    assert e % nw == 0
    ept = e // nw
    ch = _largest_divisor(ept, 128)
    nchunk = ept // ch
    npad = -(-n // (8 * _NS)) * (8 * _NS)
    rows_pt = npad // _NS

    mesh = plsc.VectorSubcoreMesh(core_axis_name="c", subcore_axis_name="s")
    out_type = [jax.ShapeDtypeStruct((_NC, npad, w), jnp.float32)]
    assert nchunk % 2 == 1 and nchunk >= 3
    scratch = [
        pltpu.VMEM((ept,), jnp.int32),
        pltpu.VMEM((ept,), jnp.int32),
        pltpu.VMEM((ch, w), jnp.float32),
        pltpu.VMEM((ch, w), jnp.float32),
        pltpu.VMEM_SHARED((npad, w), jnp.float32),
        pltpu.SemaphoreType.DMA,
        pltpu.SemaphoreType.DMA,
        pltpu.SemaphoreType.DMA,
        pltpu.SemaphoreType.DMA,
    ]
    if with_cnt:
        out_type.append(jax.ShapeDtypeStruct((_NC * npad,), jnp.float32))
        scratch += [
            pltpu.VMEM((ch,), jnp.float32),
            pltpu.VMEM((rows_pt,), jnp.float32),
            pltpu.VMEM_SHARED((npad,), jnp.float32),
        ]

    def body(y_hbm, src_hbm, dst_hbm, zer_hbm, *rest):
        if with_cnt:
            (acc_out, cnt_out, srcs_v, dsts_v, rows0, rows1,
             acc_s, semg0, semg1, semi, semz, ones_v, zc_v, cnt_s) = rest
        else:
            (acc_out, srcs_v, dsts_v, rows0, rows1, acc_s,
             semg0, semg1, semi, semz) = rest
        cid = lax.axis_index("c")
        sid = lax.axis_index("s")
        wid = sid * _NC + cid

        ebase = wid * ept
        pltpu.async_copy(src_hbm.at[pl.ds(ebase, ept)], srcs_v, semi)
        pltpu.async_copy(dst_hbm.at[pl.ds(ebase, ept)], dsts_v, semi)

        r0 = sid * rows_pt
        zsrc = zer_hbm.at[pl.ds(r0, rows_pt)]
        zdst = acc_s.at[pl.ds(r0, rows_pt)]
        pltpu.async_copy(zsrc, zdst, semz)
        if with_cnt:
            def onesfill(j, _):
                ones_v[pl.ds(j * 16, 16)] = jnp.ones((16,), jnp.float32)
                return 0

            lax.fori_loop(0, ch // 16, onesfill, 0)

            def zcfill(j, _):
                zc_v[pl.ds(j * 16, 16)] = jnp.zeros((16,), jnp.float32)
                return 0

            lax.fori_loop(0, rows_pt // 16, zcfill, 0)
            pltpu.sync_copy(zc_v, cnt_s.at[pl.ds(r0, rows_pt)])
        pltpu.make_async_copy(zsrc, zdst, semz).wait()
        plsc.subcore_barrier()

        pltpu.make_async_copy(src_hbm.at[pl.ds(ebase, ept)], srcs_v,
                              semi).wait()
        pltpu.make_async_copy(dst_hbm.at[pl.ds(ebase, ept)], dsts_v,
                              semi).wait()

        def gath(j, buf, sem):
            pltpu.async_copy(y_hbm.at[srcs_v.at[pl.ds(j * ch, ch)]],
                             buf, sem)

        def gwait(j, buf, sem):
            pltpu.make_async_copy(y_hbm.at[srcs_v.at[pl.ds(j * ch, ch)]],
                                  buf, sem).wait()

        def scat(j, buf):
            dv = dsts_v.at[pl.ds(j * ch, ch)]
            pltpu.sync_copy(buf, acc_s.at[dv], add=True)
            if with_cnt:
                pltpu.sync_copy(ones_v, cnt_s.at[dv], add=True)

        gath(0, rows0, semg0)

        def pair(k, _):
            j = 2 * k
            gath(j + 1, rows1, semg1)
            gwait(j, rows0, semg0)
            scat(j, rows0)
            gath(j + 2, rows0, semg0)
            gwait(j + 1, rows1, semg1)
            scat(j + 1, rows1)
            return 0

        lax.fori_loop(0, (nchunk - 1) // 2, pair, 0)
        gwait(nchunk - 1, rows0, semg0)
        scat(nchunk - 1, rows0)
        plsc.subcore_barrier()

        pltpu.sync_copy(acc_s.at[pl.ds(r0, rows_pt)],
                        acc_out.at[cid, pl.ds(r0, rows_pt)])
        if with_cnt:
            pltpu.sync_copy(cnt_s.at[pl.ds(r0, rows_pt)], zc_v)
            pltpu.sync_copy(zc_v, cnt_out.at[pl.ds(cid * npad + r0, rows_pt)])

    fn = pl.kernel(body, out_type=out_type, mesh=mesh, scratch_types=scratch)
    return fn(y, src, dst, jnp.zeros((npad, w), jnp.float32))


def _dot(a, b):
    return jnp.dot(a, b, preferred_element_type=jnp.float32)


def _mean_from_partials(p_ref, c_ref):
    psum = p_ref[0] + p_ref[1]
    cnt = jnp.maximum(c_ref[0] + c_ref[1], 1.0)
    return psum / cnt


def _tc_layer1(agg, cntp, x, W_l1, b_l1, W_r1, W_l2, b_l2, W_r2, interpret=False):
    n, d = x.shape
    k1 = W_l1.shape[1]
    k2 = W_l2.shape[1]
    R = 1000
    grid = (n // R,)

    def body(p_ref, c_ref, x_ref, wl1, bl1, wr1, wl2, bl2, wr2, y2_ref, s2_ref):
        mean = _mean_from_partials(p_ref, c_ref)
        h1 = jnp.maximum(
            _dot(mean, wl1[...]) + bl1[...] + _dot(x_ref[...], wr1[...]), 0.0)
        y2_ref[...] = _dot(h1, wl2[...])
        s2_ref[...] = _dot(h1, wr2[...]) + bl2[...]

    return pl.pallas_call(
        body,
        grid=grid,
        in_specs=[
            pl.BlockSpec((_NC, R, d), lambda i: (0, i, 0)),
            pl.BlockSpec((_NC, R, 1), lambda i: (0, i, 0)),
            pl.BlockSpec((R, d), lambda i: (i, 0)),
            pl.BlockSpec((d, k1), lambda i: (0, 0)),
            pl.BlockSpec((1, k1), lambda i: (0, 0)),
            pl.BlockSpec((d, k1), lambda i: (0, 0)),
            pl.BlockSpec((k1, k2), lambda i: (0, 0)),
            pl.BlockSpec((1, k2), lambda i: (0, 0)),
            pl.BlockSpec((k1, k2), lambda i: (0, 0)),
        ],
        out_specs=[
            pl.BlockSpec((R, k2), lambda i: (i, 0)),
            pl.BlockSpec((R, k2), lambda i: (i, 0)),
        ],
        out_shape=[
            jax.ShapeDtypeStruct((n, k2), jnp.float32),
            jax.ShapeDtypeStruct((n, k2), jnp.float32),
        ],
        interpret=interpret,
    )(agg, cntp, x, W_l1, b_l1.reshape(1, -1), W_r1, W_l2,
      b_l2.reshape(1, -1), W_r2)


def _tc_layer2(agg, cntp, s2, Wcat, bcat, interpret=False):
    n, d = s2.shape
    k2 = Wcat.shape[1]
    R = 1000
    grid = (n // R,)

    def body(p_ref, c_ref, s2_ref, wcat, bc, ycat_ref):
        mean = _mean_from_partials(p_ref, c_ref)
        h2 = jnp.maximum(mean + s2_ref[...], 0.0)
        ycat_ref[...] = _dot(h2, wcat[...]) + bc[...]

    return pl.pallas_call(
        body,
        grid=grid,
        in_specs=[
            pl.BlockSpec((_NC, R, d), lambda i: (0, i, 0)),
            pl.BlockSpec((_NC, R, 1), lambda i: (0, i, 0)),
            pl.BlockSpec((R, d), lambda i: (i, 0)),
            pl.BlockSpec((d, k2), lambda i: (0, 0)),
            pl.BlockSpec((1, k2), lambda i: (0, 0)),
        ],
        out_specs=pl.BlockSpec((R, k2), lambda i: (i, 0)),
        out_shape=jax.ShapeDtypeStruct((n, k2), jnp.float32),
        interpret=interpret,
    )(agg, cntp, s2, Wcat, bcat)


def _tc_layer3(agg, cntp, ycat, batch2d, W_lin, b_lin, interpret=False):
    n, dc = ycat.shape
    d = dc // 2
    out = W_lin.shape[1]
    R = 1000
    grid = (n // R,)
    last = grid[0] - 1

    def body(p_ref, c_ref, y_ref, b_ref, wlin, blin, out_ref, acc, accg):
        i = pl.program_id(0)

        @pl.when(i == 0)
        def _():
            acc[...] = jnp.zeros_like(acc)
            accg[...] = jnp.zeros_like(accg)

        psum = p_ref[0] + p_ref[1]
        cnt = jnp.maximum(c_ref[0] + c_ref[1], 1.0)
        mean = psum[:, :d] / cnt
        h3 = jnp.maximum(mean + y_ref[...][:, d:], 0.0)
        onehot = (b_ref[...] == lax.broadcasted_iota(jnp.int32, (R, _G), 1)
                  ).astype(jnp.float32)
        acc[...] += lax.dot_general(onehot, h3, (((0,), (0,)), ((), ())),
                                    preferred_element_type=jnp.float32)
        accg[...] += lax.dot_general(
            onehot, jnp.ones((R, 128), jnp.float32), (((0,), (0,)), ((), ())),
            preferred_element_type=jnp.float32)

        @pl.when(i == last)
        def _():
            pooled = acc[...] / jnp.maximum(accg[...][:, 0:1], 1.0)
            logits = _dot(pooled, wlin[...]) + blin[...]
            m = jnp.max(logits, axis=1, keepdims=True)
            lse = jnp.log(jnp.sum(jnp.exp(logits - m), axis=1, keepdims=True))
            out_ref[...] = logits - m - lse

    return pl.pallas_call(
        body,
        grid=grid,
        in_specs=[
            pl.BlockSpec((_NC, R, dc), lambda i: (0, i, 0)),
            pl.BlockSpec((_NC, R, 1), lambda i: (0, i, 0)),
            pl.BlockSpec((R, dc), lambda i: (i, 0)),
            pl.BlockSpec((R, 1), lambda i: (i, 0)),
            pl.BlockSpec((d, out), lambda i: (0, 0)),
            pl.BlockSpec((1, out), lambda i: (0, 0)),
        ],
        out_specs=pl.BlockSpec((_G, out), lambda i: (0, 0)),
        out_shape=jax.ShapeDtypeStruct((_G, out), jnp.float32),
        scratch_shapes=[
            pltpu.VMEM((_G, d), jnp.float32),
            pltpu.VMEM((_G, 128), jnp.float32),
        ],
        interpret=interpret,
    )(agg, cntp, ycat, batch2d, W_lin, b_lin.reshape(1, -1))


def kernel(x, edge_index, batch, W_l1, b_l1, W_r1, W_l2, b_l2, W_r2,
           W_l3, b_l3, W_r3, W_lin, b_lin):
    src = edge_index[0]
    dst = edge_index[1]
    agg1, cntp = _sc_aggregate(x, src, dst, with_cnt=True)
    cntp = cntp.reshape(_NC, -1, 1)
    y2, s2 = _tc_layer1(agg1, cntp, x, W_l1, b_l1, W_r1, W_l2, b_l2, W_r2)
    agg2 = _sc_aggregate(y2, src, dst, with_cnt=False)[0]
    Wcat = jnp.concatenate([W_l3, W_r3], axis=1)
    bcat = jnp.concatenate(
        [jnp.zeros_like(b_l3), b_l3]).reshape(1, -1)
    ycat = _tc_layer2(agg2, cntp, s2, Wcat, bcat)
    agg3 = _sc_aggregate(ycat, src, dst, with_cnt=False)[0]
    return _tc_layer3(agg3, cntp, ycat, batch.reshape(-1, 1), W_lin, b_lin)
